# Initial kernel scaffold; baseline (speedup 1.0000x reference)
import functools

import jax
import jax.numpy as jnp
from jax import lax
from jax.experimental import pallas as pl
from jax.experimental.pallas import tpu as pltpu
from jax.experimental.pallas import tpu_sc as plsc

N = 10000
E = 320000
D_IN = 128
HID = 128

NC = 2
NS = 16
NW = NC * NS
CH = 128
TPW = -(-E // (NW * CH))
E_PAD = NW * CH * TPW
ROWS_PER_TILE = N // NS


def _mm_body(seq_ref, wt_ref, b_ref, o_ref):
    o_ref[...] = (
        jnp.dot(seq_ref[...], wt_ref[...], preferred_element_type=jnp.float32)
        + b_ref[...]
    )


def _linear(seq, Wt, b):
    grid = 10
    blk = N // grid
    return pl.pallas_call(
        _mm_body,
        grid=(grid,),
        in_specs=[
            pl.BlockSpec((blk, D_IN), lambda i: (i, 0)),
            pl.BlockSpec((D_IN, HID), lambda i: (0, 0)),
            pl.BlockSpec((1, HID), lambda i: (0, 0)),
        ],
        out_specs=pl.BlockSpec((blk, HID), lambda i: (i, 0)),
        out_shape=jax.ShapeDtypeStruct((N, HID), jnp.float32),
    )(seq, Wt, b)


def _sc_aggregate(h, src3, dst3, w3):
    mesh = plsc.VectorSubcoreMesh(
        core_axis_name="c", subcore_axis_name="s", num_cores=NC, num_subcores=NS
    )

    @functools.partial(
        pl.kernel,
        out_type=jax.ShapeDtypeStruct((NC, N, HID), jnp.float32),
        mesh=mesh,
        scratch_types=[
            pltpu.VMEM((TPW, CH), jnp.int32),
            pltpu.VMEM((TPW, CH), jnp.int32),
            pltpu.VMEM((TPW, CH), jnp.float32),
            pltpu.VMEM((CH, HID), jnp.float32),
            pltpu.VMEM_SHARED((N, HID), jnp.float32),
        ],
    )
    def agg(h_hbm, src_hbm, dst_hbm, w_hbm, out_hbm, src_v, dst_v, w_v, buf, acc):
        c = lax.axis_index("c")
        s = lax.axis_index("s")
        wid = c * NS + s

        pltpu.sync_copy(src_hbm.at[wid], src_v)
        pltpu.sync_copy(dst_hbm.at[wid], dst_v)
        pltpu.sync_copy(w_hbm.at[wid], w_v)

        zeros16 = jnp.zeros((16,), jnp.float32)

        @pl.loop(0, CH)
        def _(r):
            row = buf.at[r]
            for g in range(HID // 16):
                row[pl.ds(g * 16, 16)] = zeros16

        base = s * ROWS_PER_TILE

        @pl.loop(0, ROWS_PER_TILE // CH)
        def _(k):
            pltpu.sync_copy(buf, acc.at[pl.ds(base + k * CH, CH)])

        rem = ROWS_PER_TILE % CH
        if rem:
            pltpu.sync_copy(
                buf.at[pl.ds(0, rem)],
                acc.at[pl.ds(base + (ROWS_PER_TILE // CH) * CH, rem)],
            )

        plsc.subcore_barrier()

        @pl.loop(0, TPW)
        def _(j):
            pltpu.sync_copy(h_hbm.at[src_v.at[j]], buf)
            wrow = w_v.at[j]

            @pl.loop(0, CH // 16)
            def _(g):
                w16 = wrow[pl.ds(g * 16, 16)]
                for r in range(16):
                    wb = jnp.take(
                        w16,
                        jnp.full((16,), r, jnp.int32),
                        mode="promise_in_bounds",
                    )
                    row = buf.at[g * 16 + r]
                    for q in range(HID // 16):
                        sl = pl.ds(q * 16, 16)
                        row[sl] = row[sl] * wb

            pltpu.sync_copy(buf, acc.at[dst_v.at[j]], add=True)

        plsc.subcore_barrier()

        out_c = out_hbm.at[c]

        @pl.loop(0, ROWS_PER_TILE // CH)
        def _(k):
            pltpu.sync_copy(
                acc.at[pl.ds(base + k * CH, CH)],
                out_c.at[pl.ds(base + k * CH, CH)],
            )

        if rem:
            off = base + (ROWS_PER_TILE // CH) * CH
            pltpu.sync_copy(acc.at[pl.ds(off, rem)], out_c.at[pl.ds(off, rem)])

    return agg(h, src3, dst3, w3)


def _comb_body(p_ref, a_ref, o_ref):
    t = p_ref[0] + p_ref[1]
    o_ref[...] = jnp.where(t >= 0, t, a_ref[0, 0] * t)


def _combine(partials, prelu_a):
    grid = 10
    blk = N // grid
    return pl.pallas_call(
        _comb_body,
        grid=(grid,),
        in_specs=[
            pl.BlockSpec((NC, blk, HID), lambda i: (0, i, 0)),
            pl.BlockSpec((1, 1), lambda i: (0, 0)),
        ],
        out_specs=pl.BlockSpec((blk, HID), lambda i: (i, 0)),
        out_shape=jax.ShapeDtypeStruct((N, HID), jnp.float32),
    )(partials, prelu_a)


def kernel(seq, W, b, prelu_a, edge_weight, edge_index):
    h = _linear(seq, W.T, b.reshape(1, HID))

    pad = E_PAD - E
    src = jnp.concatenate([edge_index[0], jnp.zeros((pad,), jnp.int32)])
    dst = jnp.concatenate([edge_index[1], jnp.zeros((pad,), jnp.int32)])
    w = jnp.concatenate([edge_weight, jnp.zeros((pad,), jnp.float32)])
    src3 = src.reshape(NW, TPW, CH)
    dst3 = dst.reshape(NW, TPW, CH)
    w3 = w.reshape(NW, TPW, CH)

    partials = _sc_aggregate(h, src3, dst3, w3)
    return _combine(partials, prelu_a.reshape(1, 1))


# speedup vs baseline: 4.4773x; 4.4773x over previous
import functools

import jax
import jax.numpy as jnp
from jax import lax
from jax.experimental import pallas as pl
from jax.experimental.pallas import tpu as pltpu
from jax.experimental.pallas import tpu_sc as plsc

N = 10000
E = 320000
D_IN = 128
HID = 128

NC = 2
NS = 16
NW = NC * NS
CH = 128
TPW = -(-E // (NW * CH))
E_PAD = NW * CH * TPW
N_PAD = 10240
ROWS_PER_TILE = N_PAD // NS


def _mm_body(seq_ref, wt_ref, b_ref, o_ref):
    o_ref[...] = (
        jnp.dot(seq_ref[...], wt_ref[...], preferred_element_type=jnp.float32)
        + b_ref[...]
    )


def _linear(seq, Wt, b):
    grid = 10
    blk = N // grid
    return pl.pallas_call(
        _mm_body,
        grid=(grid,),
        in_specs=[
            pl.BlockSpec((blk, D_IN), lambda i: (i, 0)),
            pl.BlockSpec((D_IN, HID), lambda i: (0, 0)),
            pl.BlockSpec((1, HID), lambda i: (0, 0)),
        ],
        out_specs=pl.BlockSpec((blk, HID), lambda i: (i, 0)),
        out_shape=jax.ShapeDtypeStruct((N, HID), jnp.float32),
    )(seq, Wt, b)


def _sc_aggregate(h, src3, dst3, w3):
    mesh = plsc.VectorSubcoreMesh(
        core_axis_name="c", subcore_axis_name="s", num_cores=NC, num_subcores=NS
    )

    @functools.partial(
        pl.kernel,
        out_type=jax.ShapeDtypeStruct((NC, N_PAD, HID), jnp.float32),
        mesh=mesh,
        scratch_types=[
            pltpu.VMEM((TPW, CH), jnp.int32),
            pltpu.VMEM((TPW, CH), jnp.int32),
            pltpu.VMEM((TPW, CH), jnp.float32),
            pltpu.VMEM((CH, HID), jnp.float32),
            pltpu.VMEM_SHARED((N_PAD, HID), jnp.float32),
        ],
    )
    def agg(h_hbm, src_hbm, dst_hbm, w_hbm, out_hbm, src_v, dst_v, w_v, buf, acc):
        c = lax.axis_index("c")
        s = lax.axis_index("s")
        wid = c * NS + s

        pltpu.sync_copy(src_hbm.at[wid], src_v)
        pltpu.sync_copy(dst_hbm.at[wid], dst_v)
        pltpu.sync_copy(w_hbm.at[wid], w_v)

        zeros16 = jnp.zeros((16,), jnp.float32)

        @pl.loop(0, CH)
        def _(r):
            row = buf.at[r]
            for g in range(HID // 16):
                row[pl.ds(g * 16, 16)] = zeros16

        base = s * ROWS_PER_TILE

        @pl.loop(0, ROWS_PER_TILE // CH)
        def _(k):
            pltpu.sync_copy(buf, acc.at[pl.ds(base + k * CH, CH)])

        plsc.subcore_barrier()

        @pl.loop(0, TPW)
        def _(j):
            pltpu.sync_copy(h_hbm.at[src_v.at[j]], buf)
            wrow = w_v.at[j]

            @pl.loop(0, CH // 16)
            def _(g):
                w16 = wrow[pl.ds(g * 16, 16)]
                for r in range(16):
                    wb = lax.gather(
                        w16,
                        jnp.full((16, 1), r, jnp.int32),
                        lax.GatherDimensionNumbers(
                            offset_dims=(),
                            collapsed_slice_dims=(0,),
                            start_index_map=(0,),
                        ),
                        (1,),
                        mode=lax.GatherScatterMode.PROMISE_IN_BOUNDS,
                    )
                    row = buf.at[g * 16 + r]
                    for q in range(HID // 16):
                        sl = pl.ds(q * 16, 16)
                        row[sl] = row[sl] * wb

            pltpu.sync_copy(buf, acc.at[dst_v.at[j]], add=True)

        plsc.subcore_barrier()

        out_c = out_hbm.at[c]

        @pl.loop(0, ROWS_PER_TILE // CH)
        def _(k):
            pltpu.sync_copy(
                acc.at[pl.ds(base + k * CH, CH)],
                out_c.at[pl.ds(base + k * CH, CH)],
            )

    return agg(h, src3, dst3, w3)


def _comb_body(p_ref, a_ref, o_ref):
    t = p_ref[0] + p_ref[1]
    o_ref[...] = jnp.where(t >= 0, t, a_ref[0, 0] * t)


def _combine(partials, prelu_a):
    grid = 10
    blk = N // grid
    return pl.pallas_call(
        _comb_body,
        grid=(grid,),
        in_specs=[
            pl.BlockSpec((NC, blk, HID), lambda i: (0, i, 0)),
            pl.BlockSpec((1, 1), lambda i: (0, 0)),
        ],
        out_specs=pl.BlockSpec((blk, HID), lambda i: (i, 0)),
        out_shape=jax.ShapeDtypeStruct((N, HID), jnp.float32),
    )(partials, prelu_a)


def kernel(seq, W, b, prelu_a, edge_weight, edge_index):
    h = _linear(seq, W.T, b.reshape(1, HID))

    pad = E_PAD - E
    src = jnp.concatenate([edge_index[0], jnp.zeros((pad,), jnp.int32)])
    dst = jnp.concatenate([edge_index[1], jnp.zeros((pad,), jnp.int32)])
    w = jnp.concatenate([edge_weight, jnp.zeros((pad,), jnp.float32)])
    src3 = src.reshape(NW, TPW, CH)
    dst3 = dst.reshape(NW, TPW, CH)
    w3 = w.reshape(NW, TPW, CH)

    partials = _sc_aggregate(h, src3, dst3, w3)[:, :N, :]
    return _combine(partials, prelu_a.reshape(1, 1))
